# Initial kernel scaffold; baseline (speedup 1.0000x reference)
import jax
import jax.numpy as jnp
from jax.experimental import pallas as pl


def kernel(edge_index, edge_values, input_feature, weight):
    raise NotImplementedError("write your pallas kernel here")


# speedup vs baseline: 4.6753x; 4.6753x over previous
import dataclasses
import functools

import jax
import jax.numpy as jnp
from jax import lax
from jax.experimental import pallas as pl
from jax.experimental.pallas import tpu as pltpu
from jax.experimental.pallas import tpu_sc as plsc

N_NODES = 10000
N_EDGES = 320000
D_IN = 128
D_OUT = 128

NUM_CORES = 2
NUM_SUBCORES = 16
NUM_TILES = NUM_CORES * NUM_SUBCORES
LANES = 16

CHUNK = 128
N_CHUNKS = N_EDGES // CHUNK
CHUNKS_PER_TILE = -(-N_CHUNKS // NUM_TILES)
BAND = 640
N_PAD = NUM_SUBCORES * BAND


def _matmul(x, w):
    def body(x_ref, w_ref, o_ref):
        o_ref[...] = jnp.dot(
            x_ref[...], w_ref[...], preferred_element_type=jnp.float32
        )

    return pl.pallas_call(
        body,
        out_shape=jax.ShapeDtypeStruct((N_NODES, D_OUT), jnp.float32),
    )(x, w)


def _sum_partials(p):
    def body(p_ref, o_ref):
        o_ref[...] = p_ref[0] + p_ref[1]

    return pl.pallas_call(
        body,
        out_shape=jax.ShapeDtypeStruct((N_NODES, D_OUT), jnp.float32),
    )(p)


def _sc_aggregate(support, row1d, col1d, val1d, zeros):
    mesh = plsc.VectorSubcoreMesh(
        core_axis_name="c",
        subcore_axis_name="s",
        num_cores=NUM_CORES,
        num_subcores=NUM_SUBCORES,
    )

    cp = pltpu.CompilerParams()
    if "needs_layout_passes" in pltpu.CompilerParams.__dataclass_fields__:
        cp = dataclasses.replace(cp, needs_layout_passes=False)

    @functools.partial(
        pl.kernel,
        out_type=jax.ShapeDtypeStruct(
            (NUM_CORES, NUM_SUBCORES, BAND, D_OUT), jnp.float32
        ),
        mesh=mesh,
        compiler_params=cp,
        scratch_types=[
            pltpu.VMEM((CHUNK,), jnp.int32),
            pltpu.VMEM((CHUNK,), jnp.int32),
            pltpu.VMEM((CHUNK,), jnp.float32),
            pltpu.VMEM((CHUNK, D_OUT), jnp.float32),
            pltpu.VMEM_SHARED((N_PAD, D_OUT), jnp.float32),
        ],
    )
    def k(sup_hbm, row_hbm, col_hbm, val_hbm, zero_hbm, out_hbm,
          colv, rowv, valv, rows, acc):
        cid = lax.axis_index("c")
        sid = lax.axis_index("s")
        wid = sid * NUM_CORES + cid

        band = pl.ds(sid * BAND, BAND)
        pltpu.sync_copy(zero_hbm, acc.at[band])
        plsc.subcore_barrier()

        @pl.loop(0, CHUNKS_PER_TILE)
        def _(j):
            c = wid + NUM_TILES * j

            @pl.when(c < N_CHUNKS)
            def _():
                base = c * CHUNK
                pltpu.sync_copy(col_hbm.at[pl.ds(base, CHUNK)], colv)
                pltpu.sync_copy(row_hbm.at[pl.ds(base, CHUNK)], rowv)
                pltpu.sync_copy(val_hbm.at[pl.ds(base, CHUNK)], valv)
                pltpu.sync_copy(sup_hbm.at[colv], rows)

                @pl.loop(0, CHUNK)
                def _(e):
                    vsp = plsc.load_gather(
                        valv, [jnp.full((LANES,), e, jnp.int32)]
                    )
                    for q in range(D_OUT // LANES):
                        sl = pl.ds(q * LANES, LANES)
                        rows[e, sl] = rows[e, sl] * vsp

                pltpu.sync_copy(rows, acc.at[rowv], add=True)

        plsc.subcore_barrier()
        pltpu.sync_copy(acc.at[band], out_hbm.at[cid, sid])

    return k(support, row1d, col1d, val1d, zeros)


def kernel(edge_index, edge_values, input_feature, weight):
    support = _matmul(input_feature, weight)
    row1d = edge_index[0].astype(jnp.int32)
    col1d = edge_index[1].astype(jnp.int32)
    zeros = jnp.zeros((BAND, D_OUT), jnp.float32)
    partials = _sc_aggregate(support, row1d, col1d, edge_values, zeros)
    partials = partials.reshape(NUM_CORES, N_PAD, D_OUT)[:, :N_NODES, :]
    return _sum_partials(partials)
